# Initial kernel scaffold; baseline (speedup 1.0000x reference)
import jax
import jax.numpy as jnp
from jax.experimental import pallas as pl


def kernel(x, edge_index, edge_attr, W1, b1, W3, b3, W4, b4):
    raise NotImplementedError("write your pallas kernel here")


# speedup vs baseline: 4.1817x; 4.1817x over previous
import functools

import jax
import jax.numpy as jnp
from jax import lax
from jax.experimental import pallas as pl
from jax.experimental.pallas import tpu as pltpu
from jax.experimental.pallas import tpu_sc as plsc

_N = 10000
_E_PAD = 327680
_NW = 32
_EPW = _E_PAD // _NW
_EPT = _E_PAD // 16
_CHUNK = 64
_NCHUNK = _EPW // _CHUNK
_NPAD = 10240
_ROWS_PT = _NPAD // 16
_F = 128

_mesh = plsc.VectorSubcoreMesh(core_axis_name="c", subcore_axis_name="s")
_sc_params = pltpu.CompilerParams(needs_layout_passes=False)


def _zero16f():
    return jnp.zeros((16,), jnp.float32)


_NDEG = 10240
_DSL = _NDEG // 16


def _norm_body(rows_hbm, cols_hbm, ea_hbm, w_hbm,
               ebuf_r, ebuf_c, ebuf_a, deg_v, part_v, wout_v,
               stage_sh, degf_sh):
    c = lax.axis_index("c")
    s = lax.axis_index("s")
    wid = s * 2 + c

    base1 = s * _EPT
    pltpu.sync_copy(rows_hbm.at[pl.ds(base1, _EPT)], ebuf_r)
    pltpu.sync_copy(cols_hbm.at[pl.ds(base1, _EPT)], ebuf_c)
    pltpu.sync_copy(ea_hbm.at[pl.ds(base1, _EPT)], ebuf_a)

    def _z_deg(i, _):
        deg_v[pl.ds(i * 16, 16)] = _zero16f()
        return 0
    lax.fori_loop(0, _NDEG // 16, _z_deg, 0)

    def _deg_body(i, _):
        r = ebuf_r[pl.ds(i * 16, 16)]
        cc = ebuf_c[pl.ds(i * 16, 16)]
        a = ebuf_a[pl.ds(i * 16, 16)]
        a = jnp.where(r == cc, 0.0, a)
        plsc.addupdate_scatter(deg_v, [r], a)
        return 0
    lax.fori_loop(0, _EPT // 16, _deg_body, 0)

    pltpu.sync_copy(deg_v, stage_sh.at[s])
    plsc.subcore_barrier()
    for t in range(16):
        pltpu.sync_copy(stage_sh.at[t].at[pl.ds(s * _DSL, _DSL)],
                        part_v.at[t])

    def _red_body(j, _):
        sl = pl.ds(j * 16, 16)
        d = part_v[0, sl]
        for t in range(1, 16):
            d = d + part_v[t, sl]
        xi = plsc.bitcast(d, jnp.int32)
        yi = jnp.int32(0x5F3759DF) - (xi >> 1)
        y = plsc.bitcast(yi, jnp.float32)
        for _ in range(3):
            y = y * (1.5 - 0.5 * d * y * y)
        deg_v[pl.ds(s * _DSL + j * 16, 16)] = jnp.where(d > 0, y, 0.0)
        return 0
    lax.fori_loop(0, _DSL // 16, _red_body, 0)

    pltpu.sync_copy(deg_v.at[pl.ds(s * _DSL, _DSL)],
                    degf_sh.at[pl.ds(s * _DSL, _DSL)])
    plsc.subcore_barrier()
    pltpu.sync_copy(degf_sh, deg_v)

    ebase = c * _EPW

    def _w_body(i, _):
        off = ebase + i * 16
        r = ebuf_r[pl.ds(off, 16)]
        cc = ebuf_c[pl.ds(off, 16)]
        a = ebuf_a[pl.ds(off, 16)]
        a = jnp.where(r == cc, 0.0, a)
        dr = plsc.load_gather(deg_v, [r])
        dc = plsc.load_gather(deg_v, [cc])
        wout_v[pl.ds(i * 16, 16)] = -(dr * a * dc)
        return 0
    lax.fori_loop(0, _EPW // 16, _w_body, 0)
    pltpu.sync_copy(wout_v, w_hbm.at[pl.ds(wid * _EPW, _EPW)])


def _norm_call(rows_p, cols_p, ea_p):
    return pl.kernel(
        _norm_body,
        out_type=jax.ShapeDtypeStruct((_E_PAD,), jnp.float32),
        mesh=_mesh,
        compiler_params=_sc_params,
        scratch_types=[
            pltpu.VMEM((_EPT,), jnp.int32),
            pltpu.VMEM((_EPT,), jnp.int32),
            pltpu.VMEM((_EPT,), jnp.float32),
            pltpu.VMEM((_NDEG,), jnp.float32),
            pltpu.VMEM((16, _DSL), jnp.float32),
            pltpu.VMEM((_EPW,), jnp.float32),
            pltpu.VMEM_SHARED((16, _NDEG), jnp.float32),
            pltpu.VMEM_SHARED((_NDEG,), jnp.float32),
        ],
    )(rows_p, cols_p, ea_p)


def _spmm_body(scale_blocks, y_hbm, rows2d_hbm, cols2d_hbm, w_hbm, out_hbm,
               cidx_v, gb0, gb1, gb2, rb0, rb1, rb2,
               wb0, wb1, wb2, acc_sh,
               g0, g1, g2, s0, s1, s2, m0, m1, m2):
    c = lax.axis_index("c")
    s = lax.axis_index("s")
    wid = s * 2 + c
    gbufs = (gb0, gb1, gb2)
    rbufs = (rb0, rb1, rb2)
    wbufs = (wb0, wb1, wb2)
    gsem = (g0, g1, g2)
    ssem = (s0, s1, s2)
    msem = (m0, m1, m2)

    pltpu.sync_copy(cols2d_hbm.at[pl.ds(wid * _NCHUNK, _NCHUNK)], cidx_v)

    def _z_zb(i, _):
        for f in range(_F // 16):
            gb0[i, pl.ds(f * 16, 16)] = _zero16f()
        return 0
    lax.fori_loop(0, _CHUNK, _z_zb, 0)
    row0 = s * _ROWS_PT
    for q in range(_ROWS_PT // _CHUNK):
        pltpu.sync_copy(gb0, acc_sh.at[pl.ds(row0 + q * _CHUNK, _CHUNK)])
    plsc.subcore_barrier()

    def _issue(j, b):
        pltpu.async_copy(
            rows2d_hbm.at[pl.ds(wid * _NCHUNK + j, 1)], rbufs[b], msem[b])
        pltpu.async_copy(
            w_hbm.at[pl.ds(wid * _EPW + j * _CHUNK, _CHUNK)], wbufs[b],
            msem[b])
        pltpu.async_copy(y_hbm.at[cidx_v.at[j]], gbufs[b], gsem[b])

    def _do_chunk(j, b):
        b2 = (b + 2) % 3
        pltpu.make_async_copy(y_hbm.at[cidx_v.at[j]], gbufs[b],
                              gsem[b]).wait()
        pltpu.make_async_copy(
            rows2d_hbm.at[pl.ds(0, 1)], rbufs[b], msem[b]).wait()
        pltpu.make_async_copy(
            w_hbm.at[pl.ds(0, _CHUNK)], wbufs[b], msem[b]).wait()

        gbuf = gbufs[b]

        def _e_body(k, _):
            wb = plsc.load_gather(wbufs[b], [jnp.full((16,), k, jnp.int32)])
            for f in range(scale_blocks):
                sl = pl.ds(f * 16, 16)
                gbuf[k, sl] = wb * gbuf[k, sl]
            return 0
        lax.fori_loop(0, _CHUNK, _e_body, 0)

        pltpu.async_copy(gbuf, acc_sh.at[rbufs[b].at[0]], add=True,
                         sem=ssem[b])

        @pl.when((j >= 1) & (j + 2 < _NCHUNK))
        def _():
            pltpu.make_async_copy(gbufs[b2], acc_sh.at[rbufs[b2].at[0]],
                                  sem=ssem[b2]).wait()
            _issue(j + 2, b2)

        @pl.when((j < 1) & (j + 2 < _NCHUNK))
        def _():
            _issue(j + 2, b2)

    _issue(0, 0)
    _issue(1, 1)

    def _body(i, _):
        _do_chunk(3 * i, 0)
        _do_chunk(3 * i + 1, 1)
        _do_chunk(3 * i + 2, 2)
        return 0
    lax.fori_loop(0, (_NCHUNK - 1) // 3, _body, 0)
    _do_chunk(_NCHUNK - 1, (_NCHUNK - 1) % 3)

    for b in range(3):
        pltpu.make_async_copy(gbufs[b], acc_sh.at[rbufs[b].at[0]],
                              sem=ssem[b]).wait()

    plsc.subcore_barrier()
    pltpu.sync_copy(acc_sh.at[pl.ds(row0, _ROWS_PT)],
                    out_hbm.at[c].at[pl.ds(row0, _ROWS_PT)])


def _spmm_call(y, rows2d, cols2d, w_hat, scale_blocks):
    return pl.kernel(
        functools.partial(_spmm_body, scale_blocks),
        out_type=jax.ShapeDtypeStruct((2, _NPAD, _F), jnp.float32),
        mesh=_mesh,
        compiler_params=_sc_params,
        scratch_types=(
            [pltpu.VMEM((_NCHUNK, _CHUNK), jnp.int32)]
            + [pltpu.VMEM((_CHUNK, _F), jnp.float32) for _ in range(3)]
            + [pltpu.VMEM((1, _CHUNK), jnp.int32) for _ in range(3)]
            + [pltpu.VMEM((_CHUNK,), jnp.float32) for _ in range(3)]
            + [pltpu.VMEM_SHARED((_NPAD, _F), jnp.float32)]
            + [pltpu.SemaphoreType.DMA for _ in range(9)]
        ),
    )(y, rows2d, cols2d, w_hat)


def _mm_body(h_ref, w_ref, b_ref, o_ref):
    o_ref[...] = jnp.dot(h_ref[...], w_ref[...],
                         preferred_element_type=jnp.float32) + b_ref[...]


def _mm_call(h, wcat, bcat):
    return pl.pallas_call(
        _mm_body,
        out_shape=jax.ShapeDtypeStruct((h.shape[0], wcat.shape[1]),
                                       jnp.float32),
    )(h, wcat, bcat)


def _relu_mm_body(nf, d_ref, t_ref, u_ref, w_ref, b_ref, o_ref):
    v = (t_ref[0, :_N, :nf] + t_ref[1, :_N, :nf]
         + 2.0 * (u_ref[0, :_N, :nf] + u_ref[1, :_N, :nf]))
    h = jnp.maximum(d_ref[...] + v, 0.0)
    o_ref[...] = jnp.dot(h, w_ref[...],
                         preferred_element_type=jnp.float32) + b_ref[...]


def _relu_mm_call(d, t, u, wcat, bcat):
    return pl.pallas_call(
        functools.partial(_relu_mm_body, d.shape[1]),
        out_shape=jax.ShapeDtypeStruct((d.shape[0], wcat.shape[1]),
                                       jnp.float32),
    )(d, t, u, wcat, bcat)


def _comb_body(nf, t_ref, o_ref):
    o_ref[...] = jnp.zeros(o_ref.shape, o_ref.dtype)
    o_ref[:, :nf] = t_ref[0, :, nf:2 * nf] + t_ref[1, :, nf:2 * nf]


def _comb_call(t, nf):
    return pl.pallas_call(
        functools.partial(_comb_body, nf),
        out_shape=jax.ShapeDtypeStruct((_NPAD, _F), jnp.float32),
    )(t)


def _final_body(d_ref, t_ref, u_ref, o_ref):
    z = (d_ref[...] + t_ref[0, :_N, :5] + t_ref[1, :_N, :5]
         + 2.0 * (u_ref[0, :_N, :5] + u_ref[1, :_N, :5]))
    m = jnp.max(z, axis=1, keepdims=True)
    lse = jnp.log(jnp.sum(jnp.exp(z - m), axis=1, keepdims=True)) + m
    o_ref[...] = z - lse


def _final_call(d, t, u):
    return pl.pallas_call(
        _final_body,
        out_shape=jax.ShapeDtypeStruct(d.shape, jnp.float32),
    )(d, t, u)


def kernel(x, edge_index, edge_attr, W1, b1, W3, b3, W4, b4):
    E = edge_index.shape[1]
    pad = _E_PAD - E
    rows_p = jnp.concatenate([edge_index[0], jnp.zeros((pad,), jnp.int32)])
    cols_p = jnp.concatenate([edge_index[1], jnp.zeros((pad,), jnp.int32)])
    ea_p = jnp.concatenate([edge_attr, jnp.zeros((pad,), jnp.float32)])

    w_hat = _norm_call(rows_p, cols_p, ea_p)
    rows2d = rows_p.reshape(_NW * _NCHUNK, _CHUNK)
    cols2d = cols_p.reshape(_NW * _NCHUNK, _CHUNK)

    def wcat_bcat(W, b, fpad):
        F = W.shape[2]
        Wp = jnp.pad(W, ((0, 0), (0, 0), (0, fpad - F)))
        pad128 = 128 - 2 * fpad
        wcat = jnp.concatenate(
            [Wp[1], Wp[2], jnp.zeros((W.shape[1], pad128), jnp.float32),
             W[0] - W[2]], axis=1)
        bcat = jnp.concatenate(
            [jnp.zeros((128,), jnp.float32), b])[None, :]
        return wcat, bcat

    def spmm_pair(M, fpad):
        t = _spmm_call(M[:, :128], rows2d, cols2d, w_hat, 2 * fpad // 16)
        z2 = _comb_call(t, fpad)
        u = _spmm_call(z2, rows2d, cols2d, w_hat, fpad // 16)
        return t, u

    wc, bc = wcat_bcat(W1, b1, 64)
    M = _mm_call(x, wc, bc)
    D = M[:, 128:]
    t, u = spmm_pair(M, 64)

    wc, bc = wcat_bcat(W3, b3, 64)
    M = _relu_mm_call(D, t, u, wc, bc)
    D = M[:, 128:]
    t, u = spmm_pair(M, 64)

    wc, bc = wcat_bcat(W4, b4, 16)
    M = _relu_mm_call(D, t, u, wc, bc)
    D = M[:, 128:]
    t, u = spmm_pair(M, 16)

    return _final_call(D, t, u)


# speedup vs baseline: 4.2239x; 1.0101x over previous
import functools

import jax
import jax.numpy as jnp
from jax import lax
from jax.experimental import pallas as pl
from jax.experimental.pallas import tpu as pltpu
from jax.experimental.pallas import tpu_sc as plsc

_N = 10000
_E_PAD = 327680
_NW = 32
_EPW = _E_PAD // _NW
_EPT = _E_PAD // 16
_CHUNK = 64
_NCHUNK = _EPW // _CHUNK
_NPAD = 10240
_ROWS_PT = _NPAD // 16
_F = 128

_mesh = plsc.VectorSubcoreMesh(core_axis_name="c", subcore_axis_name="s")
_sc_params = pltpu.CompilerParams(needs_layout_passes=False)


def _zero16f():
    return jnp.zeros((16,), jnp.float32)


_NDEG = 10240
_DSL = _NDEG // 16


def _norm_body(rows_hbm, cols_hbm, ea_hbm, w_hbm,
               ebuf_r, ebuf_c, ebuf_a, deg_v, part_v, wout_v,
               stage_sh, degf_sh):
    c = lax.axis_index("c")
    s = lax.axis_index("s")
    wid = s * 2 + c

    base1 = s * _EPT
    pltpu.sync_copy(rows_hbm.at[pl.ds(base1, _EPT)], ebuf_r)
    pltpu.sync_copy(cols_hbm.at[pl.ds(base1, _EPT)], ebuf_c)
    pltpu.sync_copy(ea_hbm.at[pl.ds(base1, _EPT)], ebuf_a)

    def _z_deg(i, _):
        deg_v[pl.ds(i * 16, 16)] = _zero16f()
        return 0
    lax.fori_loop(0, _NDEG // 16, _z_deg, 0)

    def _deg_body(i, _):
        r = ebuf_r[pl.ds(i * 16, 16)]
        cc = ebuf_c[pl.ds(i * 16, 16)]
        a = ebuf_a[pl.ds(i * 16, 16)]
        a = jnp.where(r == cc, 0.0, a)
        plsc.addupdate_scatter(deg_v, [r], a)
        return 0
    lax.fori_loop(0, _EPT // 16, _deg_body, 0)

    pltpu.sync_copy(deg_v, stage_sh.at[s])
    plsc.subcore_barrier()
    for t in range(16):
        pltpu.sync_copy(stage_sh.at[t].at[pl.ds(s * _DSL, _DSL)],
                        part_v.at[t])

    def _red_body(j, _):
        sl = pl.ds(j * 16, 16)
        d = part_v[0, sl]
        for t in range(1, 16):
            d = d + part_v[t, sl]
        xi = plsc.bitcast(d, jnp.int32)
        yi = jnp.int32(0x5F3759DF) - (xi >> 1)
        y = plsc.bitcast(yi, jnp.float32)
        for _ in range(3):
            y = y * (1.5 - 0.5 * d * y * y)
        deg_v[pl.ds(s * _DSL + j * 16, 16)] = jnp.where(d > 0, y, 0.0)
        return 0
    lax.fori_loop(0, _DSL // 16, _red_body, 0)

    pltpu.sync_copy(deg_v.at[pl.ds(s * _DSL, _DSL)],
                    degf_sh.at[pl.ds(s * _DSL, _DSL)])
    plsc.subcore_barrier()
    pltpu.sync_copy(degf_sh, deg_v)

    ebase = c * _EPW

    def _w_body(i, _):
        off = ebase + i * 16
        r = ebuf_r[pl.ds(off, 16)]
        cc = ebuf_c[pl.ds(off, 16)]
        a = ebuf_a[pl.ds(off, 16)]
        a = jnp.where(r == cc, 0.0, a)
        dr = plsc.load_gather(deg_v, [r])
        dc = plsc.load_gather(deg_v, [cc])
        wout_v[pl.ds(i * 16, 16)] = -(dr * a * dc)
        return 0
    lax.fori_loop(0, _EPW // 16, _w_body, 0)
    pltpu.sync_copy(wout_v, w_hbm.at[pl.ds(wid * _EPW, _EPW)])


def _norm_call(rows_p, cols_p, ea_p):
    return pl.kernel(
        _norm_body,
        out_type=jax.ShapeDtypeStruct((_E_PAD,), jnp.float32),
        mesh=_mesh,
        compiler_params=_sc_params,
        scratch_types=[
            pltpu.VMEM((_EPT,), jnp.int32),
            pltpu.VMEM((_EPT,), jnp.int32),
            pltpu.VMEM((_EPT,), jnp.float32),
            pltpu.VMEM((_NDEG,), jnp.float32),
            pltpu.VMEM((16, _DSL), jnp.float32),
            pltpu.VMEM((_EPW,), jnp.float32),
            pltpu.VMEM_SHARED((16, _NDEG), jnp.float32),
            pltpu.VMEM_SHARED((_NDEG,), jnp.float32),
        ],
    )(rows_p, cols_p, ea_p)


def _spmm_body(scale_blocks, y_hbm, rows2d_hbm, cols2d_hbm, w_hbm, out_hbm,
               cb0, cb1, cb2, cb3, cb4, gb0, gb1, gb2, gb3, gb4,
               rb0, rb1, rb2, rb3, rb4, wb0, wb1, wb2, wb3, wb4, acc_sh,
               g0, g1, g2, g3, g4, s0, s1, s2, s3, s4, m0, m1, m2, m3, m4):
    c = lax.axis_index("c")
    s = lax.axis_index("s")
    wid = s * 2 + c
    cbufs = (cb0, cb1, cb2, cb3, cb4)
    gbufs = (gb0, gb1, gb2, gb3, gb4)
    rbufs = (rb0, rb1, rb2, rb3, rb4)
    wbufs = (wb0, wb1, wb2, wb3, wb4)
    gsem = (g0, g1, g2, g3, g4)
    ssem = (s0, s1, s2, s3, s4)
    msem = (m0, m1, m2, m3, m4)

    def _z_zb(i, _):
        for f in range(_F // 16):
            gb0[i, pl.ds(f * 16, 16)] = _zero16f()
        return 0
    lax.fori_loop(0, _CHUNK, _z_zb, 0)
    row0 = s * 624
    for q in range(9):
        pltpu.sync_copy(gb0, acc_sh.at[pl.ds(row0 + q * _CHUNK, _CHUNK)])
    pltpu.sync_copy(gb0.at[pl.ds(0, 48)],
                    acc_sh.at[pl.ds(row0 + 9 * _CHUNK, 48)])

    @pl.when(s == 15)
    def _():
        pltpu.sync_copy(gb0.at[pl.ds(0, 16)], acc_sh.at[pl.ds(9984, 16)])
    plsc.subcore_barrier()

    def _issue_meta(j, b):
        pltpu.async_copy(
            cols2d_hbm.at[pl.ds(wid * _NCHUNK + j, 1)], cbufs[b], msem[b])
        pltpu.async_copy(
            rows2d_hbm.at[pl.ds(wid * _NCHUNK + j, 1)], rbufs[b], msem[b])
        pltpu.async_copy(
            w_hbm.at[pl.ds(wid * _EPW + j * _CHUNK, _CHUNK)], wbufs[b],
            msem[b])

    def _wait_meta(b):
        pltpu.make_async_copy(
            cols2d_hbm.at[pl.ds(0, 1)], cbufs[b], msem[b]).wait()
        pltpu.make_async_copy(
            rows2d_hbm.at[pl.ds(0, 1)], rbufs[b], msem[b]).wait()
        pltpu.make_async_copy(
            w_hbm.at[pl.ds(0, _CHUNK)], wbufs[b], msem[b]).wait()

    def _issue_gather(b):
        pltpu.async_copy(y_hbm.at[cbufs[b].at[0]], gbufs[b], gsem[b])

    def _do_chunk(j, b):
        b3 = (b + 3) % 5
        b4 = (b + 4) % 5

        @pl.when(j + 3 < _NCHUNK)
        def _():
            _wait_meta(b3)
            _issue_gather(b3)

        pltpu.make_async_copy(y_hbm.at[cbufs[b].at[0]], gbufs[b],
                              gsem[b]).wait()

        gbuf = gbufs[b]

        def _e_body(k, _):
            wb = plsc.load_gather(wbufs[b], [jnp.full((16,), k, jnp.int32)])
            for f in range(scale_blocks):
                sl = pl.ds(f * 16, 16)
                gbuf[k, sl] = wb * gbuf[k, sl]
            return 0
        lax.fori_loop(0, _CHUNK, _e_body, 0)

        pltpu.async_copy(gbuf, acc_sh.at[rbufs[b].at[0]], add=True,
                         sem=ssem[b])

        @pl.when((j >= 1) & (j + 4 < _NCHUNK))
        def _():
            pltpu.make_async_copy(gbufs[b4], acc_sh.at[rbufs[b4].at[0]],
                                  sem=ssem[b4]).wait()
            _issue_meta(j + 4, b4)

        @pl.when((j < 1) & (j + 4 < _NCHUNK))
        def _():
            _issue_meta(j + 4, b4)

    for b in range(4):
        _issue_meta(b, b)
    for b in range(3):
        _wait_meta(b)
        _issue_gather(b)

    def _body(i, _):
        for u in range(5):
            _do_chunk(5 * i + u, u)
        return 0
    lax.fori_loop(0, _NCHUNK // 5, _body, 0)

    for b in range(5):
        pltpu.make_async_copy(gbufs[b], acc_sh.at[rbufs[b].at[0]],
                              sem=ssem[b]).wait()

    plsc.subcore_barrier()
    pltpu.sync_copy(acc_sh.at[pl.ds(row0, 624)],
                    out_hbm.at[c].at[pl.ds(row0, 624)])

    @pl.when(s == 15)
    def _():
        pltpu.sync_copy(acc_sh.at[pl.ds(9984, 16)],
                        out_hbm.at[c].at[pl.ds(9984, 16)])


def _spmm_call(y, rows2d, cols2d, w_hat, scale_blocks):
    return pl.kernel(
        functools.partial(_spmm_body, scale_blocks),
        out_type=jax.ShapeDtypeStruct((2, _N, _F), jnp.float32),
        mesh=_mesh,
        compiler_params=_sc_params,
        scratch_types=(
            [pltpu.VMEM((1, _CHUNK), jnp.int32) for _ in range(5)]
            + [pltpu.VMEM((_CHUNK, _F), jnp.float32) for _ in range(5)]
            + [pltpu.VMEM((1, _CHUNK), jnp.int32) for _ in range(5)]
            + [pltpu.VMEM((_CHUNK,), jnp.float32) for _ in range(5)]
            + [pltpu.VMEM_SHARED((_N, _F), jnp.float32)]
            + [pltpu.SemaphoreType.DMA for _ in range(15)]
        ),
    )(y, rows2d, cols2d, w_hat)


def _mm_body(h_ref, w_ref, b_ref, o_ref):
    o_ref[...] = jnp.dot(h_ref[...], w_ref[...],
                         preferred_element_type=jnp.float32) + b_ref[...]


def _mm_call(h, wcat, bcat):
    return pl.pallas_call(
        _mm_body,
        out_shape=jax.ShapeDtypeStruct((h.shape[0], wcat.shape[1]),
                                       jnp.float32),
    )(h, wcat, bcat)


def _relu_mm_body(nf, d_ref, t_ref, u_ref, w_ref, b_ref, o_ref):
    v = (t_ref[0, :, :nf] + t_ref[1, :, :nf]
         + 2.0 * (u_ref[0, :_N, :nf] + u_ref[1, :_N, :nf]))
    h = jnp.maximum(d_ref[...] + v, 0.0)
    o_ref[...] = jnp.dot(h, w_ref[...],
                         preferred_element_type=jnp.float32) + b_ref[...]


def _relu_mm_call(d, t, u, wcat, bcat):
    return pl.pallas_call(
        functools.partial(_relu_mm_body, d.shape[1]),
        out_shape=jax.ShapeDtypeStruct((d.shape[0], wcat.shape[1]),
                                       jnp.float32),
    )(d, t, u, wcat, bcat)


def _comb_body(nf, t_ref, o_ref):
    o_ref[...] = jnp.zeros(o_ref.shape, o_ref.dtype)
    o_ref[:, :nf] = t_ref[0, :, nf:2 * nf] + t_ref[1, :, nf:2 * nf]


def _comb_call(t, nf):
    return pl.pallas_call(
        functools.partial(_comb_body, nf),
        out_shape=jax.ShapeDtypeStruct((_N, _F), jnp.float32),
    )(t)


def _final_body(d_ref, t_ref, u_ref, o_ref):
    z = (d_ref[...] + t_ref[0, :_N, :5] + t_ref[1, :_N, :5]
         + 2.0 * (u_ref[0, :_N, :5] + u_ref[1, :_N, :5]))
    m = jnp.max(z, axis=1, keepdims=True)
    lse = jnp.log(jnp.sum(jnp.exp(z - m), axis=1, keepdims=True)) + m
    o_ref[...] = z - lse


def _final_call(d, t, u):
    return pl.pallas_call(
        _final_body,
        out_shape=jax.ShapeDtypeStruct(d.shape, jnp.float32),
    )(d, t, u)


def kernel(x, edge_index, edge_attr, W1, b1, W3, b3, W4, b4):
    E = edge_index.shape[1]
    pad = _E_PAD - E
    rows_p = jnp.concatenate([edge_index[0], jnp.zeros((pad,), jnp.int32)])
    cols_p = jnp.concatenate([edge_index[1], jnp.zeros((pad,), jnp.int32)])
    ea_p = jnp.concatenate([edge_attr, jnp.zeros((pad,), jnp.float32)])

    w_hat = _norm_call(rows_p, cols_p, ea_p)
    rows2d = rows_p.reshape(_NW * _NCHUNK, _CHUNK)
    cols2d = cols_p.reshape(_NW * _NCHUNK, _CHUNK)

    def wcat_bcat(W, b, fpad):
        F = W.shape[2]
        Wp = jnp.pad(W, ((0, 0), (0, 0), (0, fpad - F)))
        pad128 = 128 - 2 * fpad
        wcat = jnp.concatenate(
            [Wp[1], Wp[2], jnp.zeros((W.shape[1], pad128), jnp.float32),
             W[0] - W[2]], axis=1)
        bcat = jnp.concatenate(
            [jnp.zeros((128,), jnp.float32), b])[None, :]
        return wcat, bcat

    def spmm_pair(M, fpad):
        t = _spmm_call(M[:, :128], rows2d, cols2d, w_hat, 2 * fpad // 16)
        z2 = _comb_call(t, fpad)
        u = _spmm_call(z2, rows2d, cols2d, w_hat, fpad // 16)
        return t, u

    wc, bc = wcat_bcat(W1, b1, 64)
    M = _mm_call(x, wc, bc)
    D = M[:, 128:]
    t, u = spmm_pair(M, 64)

    wc, bc = wcat_bcat(W3, b3, 64)
    M = _relu_mm_call(D, t, u, wc, bc)
    D = M[:, 128:]
    t, u = spmm_pair(M, 64)

    wc, bc = wcat_bcat(W4, b4, 16)
    M = _relu_mm_call(D, t, u, wc, bc)
    D = M[:, 128:]
    t, u = spmm_pair(M, 16)

    return _final_call(D, t, u)
